# Initial kernel scaffold; baseline (speedup 1.0000x reference)
import jax
import jax.numpy as jnp
from jax.experimental import pallas as pl


def kernel(cycle_curve_data, cycle_numbers, DKP_embeddings, Wg_dkp, Wg_cyc, Wg_flat, bg, Wg_out, bg_out, We, be, Wgen, bgen):
    raise NotImplementedError("write your pallas kernel here")


# speedup vs baseline: 1.4156x; 1.4156x over previous
import jax
import jax.numpy as jnp
from jax.experimental import pallas as pl
from jax.experimental.pallas import tpu as pltpu

B = 2048
L = 32
D_MODEL = 128
D_LLM = 1024
D_FF = 512
E = 8
D_IN = L * D_MODEL
EPS = 1e-09

BB = 256


def _moe_kernel(x_ref, dkp_ref, cyc_ref, Wgf_ref, Wgd_ref, Wgc_ref, bg_ref,
                Wgo_ref, bgo_ref, We_ref, be_ref, Wgen_ref, bgen_ref, out_ref):
    xb = x_ref[...].astype(jnp.bfloat16)
    dkpb = dkp_ref[...].astype(jnp.bfloat16)
    z = jnp.dot(xb, Wgf_ref[...], preferred_element_type=jnp.float32)
    z = z + jnp.dot(dkpb, Wgd_ref[...], preferred_element_type=jnp.float32)
    z = z + cyc_ref[...] * Wgc_ref[...]
    z = z + bg_ref[...]
    hb = jax.nn.gelu(z).astype(jnp.bfloat16)
    logits = jnp.dot(hb, Wgo_ref[...], preferred_element_type=jnp.float32) + bgo_ref[...]

    cols = jax.lax.broadcasted_iota(jnp.int32, (BB, E), 1)
    m1 = jnp.max(logits, axis=1, keepdims=True)
    a1 = jnp.min(jnp.where(logits == m1, cols, E), axis=1, keepdims=True)
    sel1 = cols == a1
    rest = jnp.where(sel1, -jnp.inf, logits)
    m2 = jnp.max(rest, axis=1, keepdims=True)
    a2 = jnp.min(jnp.where(rest == m2, cols, E), axis=1, keepdims=True)
    sel = sel1 | (cols == a2)

    p = jnp.exp(logits - m1)
    probs = p / jnp.sum(p, axis=1, keepdims=True)
    gated = jnp.where(sel, probs, 0.0)
    gates = gated / (jnp.sum(gated, axis=1, keepdims=True) + EPS)
    gates_b = gates.astype(jnp.bfloat16).astype(jnp.float32)

    acc = jnp.zeros((BB, D_MODEL), dtype=jnp.float32)
    for e in range(E):
        pe = jnp.dot(xb, We_ref[e], preferred_element_type=jnp.float32) + be_ref[e]
        pe_b = pe.astype(jnp.bfloat16).astype(jnp.float32)
        acc = acc + gates_b[:, e:e + 1] * pe_b
    combined = acc.astype(jnp.bfloat16).astype(jnp.float32)

    gen = jnp.dot(xb, Wgen_ref[...], preferred_element_type=jnp.float32)
    out_ref[...] = gen + bgen_ref[...] + combined


def kernel(cycle_curve_data, cycle_numbers, DKP_embeddings, Wg_dkp, Wg_cyc,
           Wg_flat, bg, Wg_out, bg_out, We, be, Wgen, bgen):
    b = cycle_curve_data.shape[0]
    flat = cycle_curve_data.reshape(b, -1)
    Wgf_b = Wg_flat.astype(jnp.bfloat16)
    Wgd_b = Wg_dkp.astype(jnp.bfloat16)
    Wgo_b = Wg_out.astype(jnp.bfloat16)
    We_b = We.astype(jnp.bfloat16)
    Wgen_b = Wgen.astype(jnp.bfloat16)
    bg2 = bg.reshape(1, -1)
    bgo2 = bg_out.reshape(1, -1)
    bgen2 = bgen.reshape(1, -1)

    grid = b // BB
    out = pl.pallas_call(
        _moe_kernel,
        grid=(grid,),
        in_specs=[
            pl.BlockSpec((BB, D_IN), lambda i: (i, 0)),
            pl.BlockSpec((BB, D_LLM), lambda i: (i, 0)),
            pl.BlockSpec((BB, 1), lambda i: (i, 0)),
            pl.BlockSpec((D_IN, D_FF), lambda i: (0, 0)),
            pl.BlockSpec((D_LLM, D_FF), lambda i: (0, 0)),
            pl.BlockSpec((1, D_FF), lambda i: (0, 0)),
            pl.BlockSpec((1, D_FF), lambda i: (0, 0)),
            pl.BlockSpec((D_FF, E), lambda i: (0, 0)),
            pl.BlockSpec((1, E), lambda i: (0, 0)),
            pl.BlockSpec((E, D_IN, D_MODEL), lambda i: (0, 0, 0)),
            pl.BlockSpec((E, D_MODEL), lambda i: (0, 0)),
            pl.BlockSpec((D_IN, D_MODEL), lambda i: (0, 0)),
            pl.BlockSpec((1, D_MODEL), lambda i: (0, 0)),
        ],
        out_specs=pl.BlockSpec((BB, D_MODEL), lambda i: (i, 0)),
        out_shape=jax.ShapeDtypeStruct((b, D_MODEL), jnp.float32),
        compiler_params=pltpu.CompilerParams(
            dimension_semantics=("arbitrary",),
        ),
    )(flat, DKP_embeddings, cycle_numbers, Wgf_b, Wgd_b, Wg_cyc, bg2,
      Wgo_b, bgo2, We_b, be, Wgen_b, bgen2)
    return (out, jnp.float32(0.0))


# speedup vs baseline: 1.6505x; 1.1660x over previous
import jax
import jax.numpy as jnp
from jax.experimental import pallas as pl
from jax.experimental.pallas import tpu as pltpu

B = 2048
L = 32
D_MODEL = 128
D_LLM = 1024
D_FF = 512
E = 8
D_IN = L * D_MODEL
EPS = 1e-09

BB = 256
N_ALL = D_FF + E * D_MODEL + D_MODEL


def _moe_kernel(x_ref, dkp_ref, cyc_ref, Wgf_ref, Wgd_ref, Wgc_ref, bg_ref,
                Wgo_ref, bgo_ref, We_ref, be_ref, Wgen_ref, bgen_ref, out_ref,
                Wall_s, Wgd_s, Wgo_s):
    i = pl.program_id(0)

    @pl.when(i == 0)
    def _cast_weights():
        Wall_s[:, 0:D_FF] = Wgf_ref[...].astype(jnp.bfloat16)
        for e in range(E):
            Wall_s[:, D_FF + e * D_MODEL:D_FF + (e + 1) * D_MODEL] = (
                We_ref[e].astype(jnp.bfloat16))
        Wall_s[:, D_FF + E * D_MODEL:] = Wgen_ref[...].astype(jnp.bfloat16)
        Wgd_s[...] = Wgd_ref[...].astype(jnp.bfloat16)
        Wgo_s[...] = Wgo_ref[...].astype(jnp.bfloat16)

    xb = x_ref[...].astype(jnp.bfloat16)
    dkpb = dkp_ref[...].astype(jnp.bfloat16)

    big = jnp.dot(xb, Wall_s[...], preferred_element_type=jnp.float32)

    z = big[:, 0:D_FF]
    z = z + jnp.dot(dkpb, Wgd_s[...], preferred_element_type=jnp.float32)
    z = z + cyc_ref[...] * Wgc_ref[...]
    z = z + bg_ref[...]
    hb = jax.nn.gelu(z).astype(jnp.bfloat16)
    logits = jnp.dot(hb, Wgo_s[...], preferred_element_type=jnp.float32) + bgo_ref[...]

    cols = jax.lax.broadcasted_iota(jnp.int32, (BB, E), 1)
    m1 = jnp.max(logits, axis=1, keepdims=True)
    a1 = jnp.min(jnp.where(logits == m1, cols, E), axis=1, keepdims=True)
    sel1 = cols == a1
    rest = jnp.where(sel1, -jnp.inf, logits)
    m2 = jnp.max(rest, axis=1, keepdims=True)
    a2 = jnp.min(jnp.where(rest == m2, cols, E), axis=1, keepdims=True)
    sel = sel1 | (cols == a2)

    p = jnp.exp(logits - m1)
    probs = p / jnp.sum(p, axis=1, keepdims=True)
    gated = jnp.where(sel, probs, 0.0)
    gates = gated / (jnp.sum(gated, axis=1, keepdims=True) + EPS)
    gates_b = gates.astype(jnp.bfloat16).astype(jnp.float32)

    acc = jnp.zeros((BB, D_MODEL), dtype=jnp.float32)
    for e in range(E):
        pe = big[:, D_FF + e * D_MODEL:D_FF + (e + 1) * D_MODEL] + be_ref[e]
        pe_b = pe.astype(jnp.bfloat16).astype(jnp.float32)
        acc = acc + gates_b[:, e:e + 1] * pe_b
    combined = acc.astype(jnp.bfloat16).astype(jnp.float32)

    gen = big[:, D_FF + E * D_MODEL:]
    out_ref[...] = gen + bgen_ref[...] + combined


def kernel(cycle_curve_data, cycle_numbers, DKP_embeddings, Wg_dkp, Wg_cyc,
           Wg_flat, bg, Wg_out, bg_out, We, be, Wgen, bgen):
    b = cycle_curve_data.shape[0]
    flat = cycle_curve_data.reshape(b, -1)
    bg2 = bg.reshape(1, -1)
    bgo2 = bg_out.reshape(1, -1)
    bgen2 = bgen.reshape(1, -1)

    grid = b // BB
    out = pl.pallas_call(
        _moe_kernel,
        grid=(grid,),
        in_specs=[
            pl.BlockSpec((BB, D_IN), lambda i: (i, 0)),
            pl.BlockSpec((BB, D_LLM), lambda i: (i, 0)),
            pl.BlockSpec((BB, 1), lambda i: (i, 0)),
            pl.BlockSpec((D_IN, D_FF), lambda i: (0, 0)),
            pl.BlockSpec((D_LLM, D_FF), lambda i: (0, 0)),
            pl.BlockSpec((1, D_FF), lambda i: (0, 0)),
            pl.BlockSpec((1, D_FF), lambda i: (0, 0)),
            pl.BlockSpec((D_FF, E), lambda i: (0, 0)),
            pl.BlockSpec((1, E), lambda i: (0, 0)),
            pl.BlockSpec((E, D_IN, D_MODEL), lambda i: (0, 0, 0)),
            pl.BlockSpec((E, D_MODEL), lambda i: (0, 0)),
            pl.BlockSpec((D_IN, D_MODEL), lambda i: (0, 0)),
            pl.BlockSpec((1, D_MODEL), lambda i: (0, 0)),
        ],
        out_specs=pl.BlockSpec((BB, D_MODEL), lambda i: (i, 0)),
        out_shape=jax.ShapeDtypeStruct((b, D_MODEL), jnp.float32),
        scratch_shapes=[
            pltpu.VMEM((D_IN, N_ALL), jnp.bfloat16),
            pltpu.VMEM((D_LLM, D_FF), jnp.bfloat16),
            pltpu.VMEM((D_FF, E), jnp.bfloat16),
        ],
        compiler_params=pltpu.CompilerParams(
            dimension_semantics=("arbitrary",),
        ),
    )(flat, DKP_embeddings, cycle_numbers, Wg_flat, Wg_dkp, Wg_cyc, bg2,
      Wg_out, bgo2, We, be, Wgen, bgen2)
    return (out, jnp.float32(0.0))


# speedup vs baseline: 2.4690x; 1.4958x over previous
import jax
import jax.numpy as jnp
from jax.experimental import pallas as pl
from jax.experimental.pallas import tpu as pltpu

B = 2048
L = 32
D_MODEL = 128
D_LLM = 1024
D_FF = 512
E = 8
D_IN = L * D_MODEL
EPS = 1e-09

BB = 256
N_ALL = D_FF + E * D_MODEL + D_MODEL


def _moe_kernel(x_ref, dkp_ref, cyc_ref, Wgf_ref, Wgd_ref, Wgc_ref, bg_ref,
                Wgo_ref, bgo_ref, We_ref, be_ref, Wgen_ref, bgen_ref, out_ref,
                Wall_s, Wgd_s, Wgo_s):
    i = pl.program_id(0)

    @pl.when(i == 0)
    def _cast_weights():
        Wall_s[:, 0:D_FF] = Wgf_ref[...].astype(jnp.bfloat16)
        for e in range(E):
            Wall_s[:, D_FF + e * D_MODEL:D_FF + (e + 1) * D_MODEL] = (
                We_ref[e].astype(jnp.bfloat16))
        Wall_s[:, D_FF + E * D_MODEL:] = Wgen_ref[...].astype(jnp.bfloat16)
        Wgd_s[...] = Wgd_ref[...].astype(jnp.bfloat16)
        Wgo_s[...] = Wgo_ref[...].astype(jnp.bfloat16)

    xb = x_ref[...].reshape(BB, D_IN).astype(jnp.bfloat16)
    dkpb = dkp_ref[...].astype(jnp.bfloat16)

    big = jnp.dot(xb, Wall_s[...], preferred_element_type=jnp.float32)

    z = big[:, 0:D_FF]
    z = z + jnp.dot(dkpb, Wgd_s[...], preferred_element_type=jnp.float32)
    z = z + cyc_ref[...] * Wgc_ref[...]
    z = z + bg_ref[...]
    hb = jax.nn.gelu(z).astype(jnp.bfloat16)
    logits = jnp.dot(hb, Wgo_s[...], preferred_element_type=jnp.float32) + bgo_ref[...]

    cols = jax.lax.broadcasted_iota(jnp.int32, (BB, E), 1)
    m1 = jnp.max(logits, axis=1, keepdims=True)
    a1 = jnp.min(jnp.where(logits == m1, cols, E), axis=1, keepdims=True)
    sel1 = cols == a1
    rest = jnp.where(sel1, -jnp.inf, logits)
    m2 = jnp.max(rest, axis=1, keepdims=True)
    a2 = jnp.min(jnp.where(rest == m2, cols, E), axis=1, keepdims=True)
    sel = sel1 | (cols == a2)

    p = jnp.exp(logits - m1)
    probs = p / jnp.sum(p, axis=1, keepdims=True)
    gated = jnp.where(sel, probs, 0.0)
    gates = gated / (jnp.sum(gated, axis=1, keepdims=True) + EPS)
    gates_b = gates.astype(jnp.bfloat16).astype(jnp.float32)

    acc = jnp.zeros((BB, D_MODEL), dtype=jnp.float32)
    for e in range(E):
        pe = big[:, D_FF + e * D_MODEL:D_FF + (e + 1) * D_MODEL] + be_ref[e]
        pe_b = pe.astype(jnp.bfloat16).astype(jnp.float32)
        acc = acc + gates_b[:, e:e + 1] * pe_b
    combined = acc.astype(jnp.bfloat16).astype(jnp.float32)

    gen = big[:, D_FF + E * D_MODEL:]
    out_ref[...] = gen + bgen_ref[...] + combined


def kernel(cycle_curve_data, cycle_numbers, DKP_embeddings, Wg_dkp, Wg_cyc,
           Wg_flat, bg, Wg_out, bg_out, We, be, Wgen, bgen):
    b = cycle_curve_data.shape[0]
    bg2 = bg.reshape(1, -1)
    bgo2 = bg_out.reshape(1, -1)
    bgen2 = bgen.reshape(1, -1)

    grid = b // BB
    out = pl.pallas_call(
        _moe_kernel,
        grid=(grid,),
        in_specs=[
            pl.BlockSpec((BB, L, D_MODEL), lambda i: (i, 0, 0)),
            pl.BlockSpec((BB, D_LLM), lambda i: (i, 0)),
            pl.BlockSpec((BB, 1), lambda i: (i, 0)),
            pl.BlockSpec((D_IN, D_FF), lambda i: (0, 0)),
            pl.BlockSpec((D_LLM, D_FF), lambda i: (0, 0)),
            pl.BlockSpec((1, D_FF), lambda i: (0, 0)),
            pl.BlockSpec((1, D_FF), lambda i: (0, 0)),
            pl.BlockSpec((D_FF, E), lambda i: (0, 0)),
            pl.BlockSpec((1, E), lambda i: (0, 0)),
            pl.BlockSpec((E, D_IN, D_MODEL), lambda i: (0, 0, 0)),
            pl.BlockSpec((E, D_MODEL), lambda i: (0, 0)),
            pl.BlockSpec((D_IN, D_MODEL), lambda i: (0, 0)),
            pl.BlockSpec((1, D_MODEL), lambda i: (0, 0)),
        ],
        out_specs=pl.BlockSpec((BB, D_MODEL), lambda i: (i, 0)),
        out_shape=jax.ShapeDtypeStruct((b, D_MODEL), jnp.float32),
        scratch_shapes=[
            pltpu.VMEM((D_IN, N_ALL), jnp.bfloat16),
            pltpu.VMEM((D_LLM, D_FF), jnp.bfloat16),
            pltpu.VMEM((D_FF, E), jnp.bfloat16),
        ],
        compiler_params=pltpu.CompilerParams(
            dimension_semantics=("arbitrary",),
        ),
    )(cycle_curve_data, DKP_embeddings, cycle_numbers, Wg_flat, Wg_dkp, Wg_cyc, bg2,
      Wg_out, bgo2, We, be, Wgen, bgen2)
    return (out, jnp.float32(0.0))
